# Initial kernel scaffold; baseline (speedup 1.0000x reference)
import jax
import jax.numpy as jnp
from jax.experimental import pallas as pl


def kernel(x, edge_index, W1, b1, W2, b2):
    raise NotImplementedError("write your pallas kernel here")


# speedup vs baseline: 8.7166x; 8.7166x over previous
import jax
import jax.numpy as jnp
from jax import lax
from jax.experimental import pallas as pl
from jax.experimental.pallas import tpu as pltpu
from jax.experimental.pallas import tpu_sc as plsc

N = 10000
D = 128
DH = D // 2
E = 320000
K = 10
ALPHA = 0.1

NT = 16
N_PAD = 10240
RPT = N_PAD // NT
BLK = 32
NBLK = RPT // BLK
CHUNK = 128
E_FULL = E + N
CPT = (-(-E_FULL // (NT * CHUNK)) + 7) // 8 * 8
E_PAD = NT * CPT * CHUNK


def _mlp_body(x_ref, w1_ref, b1_ref, w2_ref, b2_ref, o_ref):
    h = jnp.dot(x_ref[...], w1_ref[...], preferred_element_type=jnp.float32)
    h = jnp.maximum(h + b1_ref[...], 0.0)
    o = jnp.dot(h, w2_ref[...], preferred_element_type=jnp.float32)
    o_ref[...] = o + b2_ref[...]


def _mlp(x_pad, W1, b1, W2, b2):
    BM = 1024
    return pl.pallas_call(
        _mlp_body,
        grid=(N_PAD // BM,),
        in_specs=[
            pl.BlockSpec((BM, D), lambda i: (i, 0)),
            pl.BlockSpec((D, D), lambda i: (0, 0)),
            pl.BlockSpec((1, D), lambda i: (0, 0)),
            pl.BlockSpec((D, D), lambda i: (0, 0)),
            pl.BlockSpec((1, D), lambda i: (0, 0)),
        ],
        out_specs=pl.BlockSpec((BM, D), lambda i: (i, 0)),
        out_shape=jax.ShapeDtypeStruct((N_PAD, D), jnp.float32),
    )(x_pad, W1, b1.reshape(1, D), W2, b2.reshape(1, D))


def _sc_body(h_hbm, edges_hbm, ones_hbm, zrow_hbm, z16_hbm,
             z_hbm, u_hbm,
             agg, dinv_sp, idx_a, idx_b, gb_a, gb_b, row_a, row_b,
             ones_vm, zb, zb16, blk, hblk, dvblk, sem_a, sem_b):
    cid = lax.axis_index("c")
    tid = lax.axis_index("s")
    ebase = tid * CPT
    rbase = tid * RPT
    hbase = cid * N_PAD + rbase
    bias = (cid * N_PAD).astype(jnp.int32)

    def load_idx(j, idx, gb):
        pltpu.sync_copy(edges_hbm.at[ebase + j], idx)
        for c in range(CHUNK // 16):
            sl = pl.ds(c * 16, 16)
            gb[sl] = idx[0, sl] + bias

    pltpu.sync_copy(ones_hbm, ones_vm)
    pltpu.sync_copy(zrow_hbm, zb)
    pltpu.sync_copy(z16_hbm, zb16)

    @pl.loop(0, NBLK)
    def _zero(b):
        off = rbase + b * BLK
        pltpu.sync_copy(zb16, dinv_sp.at[pl.ds(off, BLK)])
        pltpu.sync_copy(zb, agg.at[pl.ds(off, BLK)])

    plsc.subcore_barrier()

    @pl.loop(0, CPT)
    def _deg(j):
        pltpu.sync_copy(edges_hbm.at[ebase + j], idx_a)
        pltpu.sync_copy(ones_vm, dinv_sp.at[idx_a.at[1]], add=True)

    plsc.subcore_barrier()

    @pl.loop(0, NBLK)
    def _dinv(b):
        off = rbase + b * BLK
        pltpu.sync_copy(dinv_sp.at[pl.ds(off, BLK)], dvblk)

        @pl.loop(0, BLK // 16)
        def _grp(g):
            for i in range(16):
                r = g * 16 + i
                d = dvblk[r, :]
                di = lax.bitcast_convert_type(d, jnp.int32)
                y = lax.bitcast_convert_type(
                    jnp.int32(0x5F3759DF) - (di >> 1), jnp.float32)
                for _ in range(3):
                    y = y * (1.5 - 0.5 * d * y * y)
                dvblk[r, :] = y

        pltpu.sync_copy(dvblk, dinv_sp.at[pl.ds(off, BLK)])

    @pl.loop(0, NBLK)
    def _u0(b):
        off = rbase + b * BLK
        pltpu.sync_copy(h_hbm.at[pl.ds(hbase + b * BLK, BLK)], hblk)
        pltpu.sync_copy(dinv_sp.at[pl.ds(off, BLK)], dvblk)

        @pl.loop(0, BLK)
        def _row(r):
            dvec = dvblk[r, :]
            for c in range(DH // 16):
                sl = pl.ds(c * 16, 16)
                hblk[r, sl] = hblk[r, sl] * dvec

        pltpu.sync_copy(hblk, u_hbm.at[pl.ds(hbase + b * BLK, BLK)])

    plsc.subcore_barrier()

    @pl.loop(0, K)
    def _iter(_k):
        @pl.loop(0, CPT, step=2)
        def _edges(j):
            load_idx(j, idx_a, gb_a)
            load_idx(j + 1, idx_b, gb_b)
            cp_a = pltpu.async_copy(u_hbm.at[gb_a], row_a, sem_a)
            cp_b = pltpu.async_copy(u_hbm.at[gb_b], row_b, sem_b)
            cp_a.wait()
            pltpu.sync_copy(row_a, agg.at[idx_a.at[1]], add=True)
            cp_b.wait()
            pltpu.sync_copy(row_b, agg.at[idx_b.at[1]], add=True)

        plsc.subcore_barrier()

        @pl.loop(0, NBLK)
        def _upd(b):
            off = rbase + b * BLK
            pltpu.sync_copy(agg.at[pl.ds(off, BLK)], blk)
            pltpu.sync_copy(h_hbm.at[pl.ds(hbase + b * BLK, BLK)], hblk)
            pltpu.sync_copy(dinv_sp.at[pl.ds(off, BLK)], dvblk)

            @pl.loop(0, BLK)
            def _row(r):
                dvec = dvblk[r, :]
                svec = (1.0 - ALPHA) * dvec
                for c in range(DH // 16):
                    sl = pl.ds(c * 16, 16)
                    z = blk[r, sl] * svec + ALPHA * hblk[r, sl]
                    blk[r, sl] = z * dvec

            pltpu.sync_copy(blk, u_hbm.at[pl.ds(hbase + b * BLK, BLK)])
            pltpu.sync_copy(zb, agg.at[pl.ds(off, BLK)])

        plsc.subcore_barrier()

    @pl.loop(0, NBLK)
    def _fin(b):
        off = rbase + b * BLK
        pltpu.sync_copy(u_hbm.at[pl.ds(hbase + b * BLK, BLK)], hblk)
        pltpu.sync_copy(dinv_sp.at[pl.ds(off, BLK)], dvblk)

        @pl.loop(0, BLK)
        def _row(r):
            rvec = 1.0 / dvblk[r, :]
            for c in range(DH // 16):
                sl = pl.ds(c * 16, 16)
                hblk[r, sl] = hblk[r, sl] * rvec

        pltpu.sync_copy(hblk, z_hbm.at[pl.ds(hbase + b * BLK, BLK)])


_sc_call = pl.kernel(
    _sc_body,
    out_type=(jax.ShapeDtypeStruct((2 * N_PAD, DH), jnp.float32),
              jax.ShapeDtypeStruct((2 * N_PAD, DH), jnp.float32)),
    mesh=plsc.VectorSubcoreMesh(core_axis_name="c", subcore_axis_name="s"),
    compiler_params=pltpu.CompilerParams(use_tc_tiling_on_sc=False),
    scratch_types=[
        pltpu.VMEM_SHARED((N_PAD, DH), jnp.float32),
        pltpu.VMEM_SHARED((N_PAD, 16), jnp.float32),
        pltpu.VMEM((2, CHUNK), jnp.int32),
        pltpu.VMEM((2, CHUNK), jnp.int32),
        pltpu.VMEM((CHUNK,), jnp.int32),
        pltpu.VMEM((CHUNK,), jnp.int32),
        pltpu.VMEM((CHUNK, DH), jnp.float32),
        pltpu.VMEM((CHUNK, DH), jnp.float32),
        pltpu.VMEM((CHUNK, 16), jnp.float32),
        pltpu.VMEM((BLK, DH), jnp.float32),
        pltpu.VMEM((BLK, 16), jnp.float32),
        pltpu.VMEM((BLK, DH), jnp.float32),
        pltpu.VMEM((BLK, DH), jnp.float32),
        pltpu.VMEM((BLK, 16), jnp.float32),
        pltpu.SemaphoreType.DMA,
        pltpu.SemaphoreType.DMA,
    ],
)


def kernel(x, edge_index, W1, b1, W2, b2):
    x_pad = jnp.pad(x, ((0, N_PAD - N), (0, 0)))
    h_pad = _mlp(x_pad, W1, b1, W2, b2)
    h_split = jnp.concatenate([h_pad[:, :DH], h_pad[:, DH:]], axis=0)

    src = edge_index[0]
    dst = edge_index[1]
    loop_idx = jnp.arange(N, dtype=jnp.int32)
    pad_idx = N + (jnp.arange(E_PAD - E_FULL, dtype=jnp.int32) % 16)
    srcs = jnp.concatenate([src, loop_idx, pad_idx]).reshape(NT * CPT, CHUNK)
    dsts = jnp.concatenate([dst, loop_idx, pad_idx]).reshape(NT * CPT, CHUNK)
    edges = jnp.stack([srcs, dsts], axis=1)

    ones16 = jnp.ones((CHUNK, 16), jnp.float32)
    zrow = jnp.zeros((BLK, DH), jnp.float32)
    z16 = jnp.zeros((BLK, 16), jnp.float32)

    z_split, _ = _sc_call(h_split, edges, ones16, zrow, z16)
    return jnp.concatenate([z_split[:N], z_split[N_PAD:N_PAD + N]], axis=1)


# speedup vs baseline: 11.3877x; 1.3064x over previous
import jax
import jax.numpy as jnp
from jax import lax
from jax.experimental import pallas as pl
from jax.experimental.pallas import tpu as pltpu
from jax.experimental.pallas import tpu_sc as plsc

N = 10000
D = 128
DH = D // 2
E = 320000
K = 10
ALPHA = 0.1

NT = 16
N_PAD = 10240
RPT = N_PAD // NT
BLK = 32
NBLK = RPT // BLK
CHUNK = 128
E_FULL = E + N
CPT = (-(-E_FULL // (NT * CHUNK)) + 7) // 8 * 8
E_PAD = NT * CPT * CHUNK


def _mlp_body(x_ref, w1_ref, b1_ref, w2_ref, b2_ref, o_ref):
    h = jnp.dot(x_ref[...], w1_ref[...], preferred_element_type=jnp.float32)
    h = jnp.maximum(h + b1_ref[...], 0.0)
    o = jnp.dot(h, w2_ref[...], preferred_element_type=jnp.float32)
    o_ref[...] = o + b2_ref[...]


def _mlp(x_pad, W1, b1, W2, b2):
    BM = 1024
    return pl.pallas_call(
        _mlp_body,
        grid=(N_PAD // BM,),
        in_specs=[
            pl.BlockSpec((BM, D), lambda i: (i, 0)),
            pl.BlockSpec((D, D), lambda i: (0, 0)),
            pl.BlockSpec((1, D), lambda i: (0, 0)),
            pl.BlockSpec((D, D), lambda i: (0, 0)),
            pl.BlockSpec((1, D), lambda i: (0, 0)),
        ],
        out_specs=pl.BlockSpec((BM, D), lambda i: (i, 0)),
        out_shape=jax.ShapeDtypeStruct((N_PAD, D), jnp.float32),
    )(x_pad, W1, b1.reshape(1, D), W2, b2.reshape(1, D))


def _sc_body(h_hbm, gsrcs_hbm, dsts_hbm, ones_hbm, zrow_hbm, z16_hbm,
             z_hbm, u_hbm,
             agg, dinv_sp, src_vm, dst_a, dst_b, row_a, row_b,
             ones_vm, zb, zb16, blk, hblk, dvblk, sem_a, sem_b):
    cid = lax.axis_index("c")
    tid = lax.axis_index("s")
    ebase = tid * CPT
    rbase = tid * RPT
    hbase = cid * N_PAD + rbase

    pltpu.sync_copy(gsrcs_hbm.at[pl.ds(cid * NT * CPT + ebase, CPT)], src_vm)
    pltpu.sync_copy(ones_hbm, ones_vm)
    pltpu.sync_copy(zrow_hbm, zb)
    pltpu.sync_copy(z16_hbm, zb16)

    @pl.loop(0, NBLK)
    def _zero(b):
        off = rbase + b * BLK
        pltpu.sync_copy(zb16, dinv_sp.at[pl.ds(off, BLK)])
        pltpu.sync_copy(zb, agg.at[pl.ds(off, BLK)])

    plsc.subcore_barrier()

    @pl.loop(0, CPT)
    def _deg(j):
        pltpu.sync_copy(dsts_hbm.at[pl.ds(ebase + j, 1)], dst_a)
        pltpu.sync_copy(ones_vm, dinv_sp.at[dst_a.at[0]], add=True)

    plsc.subcore_barrier()

    @pl.loop(0, NBLK)
    def _dinv(b):
        off = rbase + b * BLK
        pltpu.sync_copy(dinv_sp.at[pl.ds(off, BLK)], dvblk)

        @pl.loop(0, BLK // 16)
        def _grp(g):
            for i in range(16):
                r = g * 16 + i
                d = dvblk[r, :]
                di = lax.bitcast_convert_type(d, jnp.int32)
                y = lax.bitcast_convert_type(
                    jnp.int32(0x5F3759DF) - (di >> 1), jnp.float32)
                for _ in range(3):
                    y = y * (1.5 - 0.5 * d * y * y)
                dvblk[r, :] = y

        pltpu.sync_copy(dvblk, dinv_sp.at[pl.ds(off, BLK)])

    @pl.loop(0, NBLK)
    def _u0(b):
        off = rbase + b * BLK
        pltpu.sync_copy(h_hbm.at[pl.ds(hbase + b * BLK, BLK)], hblk)
        pltpu.sync_copy(dinv_sp.at[pl.ds(off, BLK)], dvblk)

        @pl.loop(0, BLK)
        def _row(r):
            dvec = dvblk[r, :]
            for c in range(DH // 16):
                sl = pl.ds(c * 16, 16)
                hblk[r, sl] = hblk[r, sl] * dvec

        pltpu.sync_copy(hblk, u_hbm.at[pl.ds(hbase + b * BLK, BLK)])

    plsc.subcore_barrier()

    @pl.loop(0, K)
    def _iter(_k):
        @pl.loop(0, CPT, step=2)
        def _edges(j):
            cp_a = pltpu.async_copy(u_hbm.at[src_vm.at[j]], row_a, sem_a)
            cp_b = pltpu.async_copy(u_hbm.at[src_vm.at[j + 1]], row_b, sem_b)
            pltpu.sync_copy(dsts_hbm.at[pl.ds(ebase + j, 1)], dst_a)
            pltpu.sync_copy(dsts_hbm.at[pl.ds(ebase + j + 1, 1)], dst_b)
            cp_a.wait()
            pltpu.sync_copy(row_a, agg.at[dst_a.at[0]], add=True)
            cp_b.wait()
            pltpu.sync_copy(row_b, agg.at[dst_b.at[0]], add=True)

        plsc.subcore_barrier()

        @pl.loop(0, NBLK)
        def _upd(b):
            off = rbase + b * BLK
            pltpu.sync_copy(agg.at[pl.ds(off, BLK)], blk)
            pltpu.sync_copy(h_hbm.at[pl.ds(hbase + b * BLK, BLK)], hblk)
            pltpu.sync_copy(dinv_sp.at[pl.ds(off, BLK)], dvblk)

            @pl.loop(0, BLK)
            def _row(r):
                dvec = dvblk[r, :]
                svec = (1.0 - ALPHA) * dvec
                for c in range(DH // 16):
                    sl = pl.ds(c * 16, 16)
                    z = blk[r, sl] * svec + ALPHA * hblk[r, sl]
                    blk[r, sl] = z * dvec

            pltpu.sync_copy(blk, u_hbm.at[pl.ds(hbase + b * BLK, BLK)])
            pltpu.sync_copy(zb, agg.at[pl.ds(off, BLK)])

        plsc.subcore_barrier()

    @pl.loop(0, NBLK)
    def _fin(b):
        off = rbase + b * BLK
        pltpu.sync_copy(u_hbm.at[pl.ds(hbase + b * BLK, BLK)], hblk)
        pltpu.sync_copy(dinv_sp.at[pl.ds(off, BLK)], dvblk)

        @pl.loop(0, BLK)
        def _row(r):
            rvec = 1.0 / dvblk[r, :]
            for c in range(DH // 16):
                sl = pl.ds(c * 16, 16)
                hblk[r, sl] = hblk[r, sl] * rvec

        pltpu.sync_copy(hblk, z_hbm.at[pl.ds(hbase + b * BLK, BLK)])


_sc_call = pl.kernel(
    _sc_body,
    out_type=(jax.ShapeDtypeStruct((2 * N_PAD, DH), jnp.float32),
              jax.ShapeDtypeStruct((2 * N_PAD, DH), jnp.float32)),
    mesh=plsc.VectorSubcoreMesh(core_axis_name="c", subcore_axis_name="s"),
    compiler_params=pltpu.CompilerParams(use_tc_tiling_on_sc=False),
    scratch_types=[
        pltpu.VMEM_SHARED((N_PAD, DH), jnp.float32),
        pltpu.VMEM_SHARED((N_PAD, 16), jnp.float32),
        pltpu.VMEM((CPT, CHUNK), jnp.int32),
        pltpu.VMEM((1, CHUNK), jnp.int32),
        pltpu.VMEM((1, CHUNK), jnp.int32),
        pltpu.VMEM((CHUNK, DH), jnp.float32),
        pltpu.VMEM((CHUNK, DH), jnp.float32),
        pltpu.VMEM((CHUNK, 16), jnp.float32),
        pltpu.VMEM((BLK, DH), jnp.float32),
        pltpu.VMEM((BLK, 16), jnp.float32),
        pltpu.VMEM((BLK, DH), jnp.float32),
        pltpu.VMEM((BLK, DH), jnp.float32),
        pltpu.VMEM((BLK, 16), jnp.float32),
        pltpu.SemaphoreType.DMA,
        pltpu.SemaphoreType.DMA,
    ],
)


def kernel(x, edge_index, W1, b1, W2, b2):
    x_pad = jnp.pad(x, ((0, N_PAD - N), (0, 0)))
    h_pad = _mlp(x_pad, W1, b1, W2, b2)
    h_split = jnp.concatenate([h_pad[:, :DH], h_pad[:, DH:]], axis=0)

    src = edge_index[0]
    dst = edge_index[1]
    loop_idx = jnp.arange(N, dtype=jnp.int32)
    pad_idx = N + (jnp.arange(E_PAD - E_FULL, dtype=jnp.int32) % 16)
    srcs = jnp.concatenate([src, loop_idx, pad_idx]).reshape(NT * CPT, CHUNK)
    dsts = jnp.concatenate([dst, loop_idx, pad_idx]).reshape(NT * CPT, CHUNK)
    gsrcs = jnp.concatenate([srcs, srcs + N_PAD], axis=0)

    ones16 = jnp.ones((CHUNK, 16), jnp.float32)
    zrow = jnp.zeros((BLK, DH), jnp.float32)
    z16 = jnp.zeros((BLK, 16), jnp.float32)

    z_split, _ = _sc_call(h_split, gsrcs, dsts, ones16, zrow, z16)
    return jnp.concatenate([z_split[:N], z_split[N_PAD:N_PAD + N]], axis=1)


# speedup vs baseline: 13.4766x; 1.1834x over previous
import jax
import jax.numpy as jnp
from jax import lax
from jax.experimental import pallas as pl
from jax.experimental.pallas import tpu as pltpu
from jax.experimental.pallas import tpu_sc as plsc

N = 10000
D = 128
DH = D // 2
E = 320000
K = 10
ALPHA = 0.1

NT = 16
N_PAD = 10240
RPT = N_PAD // NT
BLK = 32
NBLK = RPT // BLK
CHUNK = 128
E_FULL = E + N
CPT = (-(-E_FULL // (NT * CHUNK)) + 7) // 8 * 8
E_PAD = NT * CPT * CHUNK


def _mlp_body(x_ref, w1_ref, b1_ref, w2_ref, b2_ref, o_ref):
    h = jnp.dot(x_ref[...], w1_ref[...], preferred_element_type=jnp.float32)
    h = jnp.maximum(h + b1_ref[...], 0.0)
    o = jnp.dot(h, w2_ref[...], preferred_element_type=jnp.float32)
    o_ref[...] = o + b2_ref[...]


def _mlp(x_pad, W1, b1, W2, b2):
    BM = 1024
    return pl.pallas_call(
        _mlp_body,
        grid=(N_PAD // BM,),
        in_specs=[
            pl.BlockSpec((BM, D), lambda i: (i, 0)),
            pl.BlockSpec((D, D), lambda i: (0, 0)),
            pl.BlockSpec((1, D), lambda i: (0, 0)),
            pl.BlockSpec((D, D), lambda i: (0, 0)),
            pl.BlockSpec((1, D), lambda i: (0, 0)),
        ],
        out_specs=pl.BlockSpec((BM, D), lambda i: (i, 0)),
        out_shape=jax.ShapeDtypeStruct((N_PAD, D), jnp.float32),
    )(x_pad, W1, b1.reshape(1, D), W2, b2.reshape(1, D))


NBUF = 4


def _sc_body(h_hbm, gsrcs_hbm, dsts_hbm, ones_hbm, zrow_hbm, z16_hbm,
             z_hbm, u_hbm,
             agg, dinv_sp, src_vm, dst_vms, row_vms,
             ones_vm, zb, zb16, blk, hblk, dvblk, gsems, ssems):
    cid = lax.axis_index("c")
    tid = lax.axis_index("s")
    ebase = tid * CPT
    rbase = tid * RPT
    hbase = cid * N_PAD + rbase

    pltpu.sync_copy(gsrcs_hbm.at[pl.ds(cid * NT * CPT + ebase, CPT)], src_vm)
    pltpu.sync_copy(ones_hbm, ones_vm)
    pltpu.sync_copy(zrow_hbm, zb)
    pltpu.sync_copy(z16_hbm, zb16)

    @pl.loop(0, NBLK)
    def _zero(b):
        off = rbase + b * BLK
        pltpu.sync_copy(zb16, dinv_sp.at[pl.ds(off, BLK)])
        pltpu.sync_copy(zb, agg.at[pl.ds(off, BLK)])

    plsc.subcore_barrier()

    @pl.loop(0, CPT, step=NBUF)
    def _deg(j):
        for b in range(NBUF):
            pltpu.sync_copy(dsts_hbm.at[pl.ds(ebase + j + b, 1)], dst_vms[b])
        cps = [pltpu.async_copy(ones_vm, dinv_sp.at[dst_vms[b].at[0]],
                                ssems[b], add=True) for b in range(NBUF)]
        for cp in cps:
            cp.wait()

    plsc.subcore_barrier()

    @pl.loop(0, NBLK)
    def _dinv(b):
        off = rbase + b * BLK
        pltpu.sync_copy(dinv_sp.at[pl.ds(off, BLK)], dvblk)

        @pl.loop(0, BLK // 16)
        def _grp(g):
            for i in range(16):
                r = g * 16 + i
                d = dvblk[r, :]
                di = lax.bitcast_convert_type(d, jnp.int32)
                y = lax.bitcast_convert_type(
                    jnp.int32(0x5F3759DF) - (di >> 1), jnp.float32)
                for _ in range(3):
                    y = y * (1.5 - 0.5 * d * y * y)
                dvblk[r, :] = y

        pltpu.sync_copy(dvblk, dinv_sp.at[pl.ds(off, BLK)])

    @pl.loop(0, NBLK)
    def _u0(b):
        off = rbase + b * BLK
        pltpu.sync_copy(h_hbm.at[pl.ds(hbase + b * BLK, BLK)], hblk)
        pltpu.sync_copy(dinv_sp.at[pl.ds(off, BLK)], dvblk)

        @pl.loop(0, BLK)
        def _row(r):
            dvec = dvblk[r, :]
            for c in range(DH // 16):
                sl = pl.ds(c * 16, 16)
                hblk[r, sl] = hblk[r, sl] * dvec

        pltpu.sync_copy(hblk, u_hbm.at[pl.ds(hbase + b * BLK, BLK)])

    plsc.subcore_barrier()

    @pl.loop(0, K)
    def _iter(_k):
        for b in range(NBUF):
            pltpu.async_copy(u_hbm.at[src_vm.at[b]], row_vms[b], gsems[b])

        @pl.loop(0, CPT, step=NBUF)
        def _edges(j):
            for b in range(NBUF):
                pltpu.sync_copy(dsts_hbm.at[pl.ds(ebase + j + b, 1)],
                                dst_vms[b])
            for b in range(NBUF):
                pltpu.make_async_copy(u_hbm.at[src_vm.at[j + b]],
                                      row_vms[b], gsems[b]).wait()
                pltpu.async_copy(row_vms[b], agg.at[dst_vms[b].at[0]],
                                 ssems[b], add=True)

            @pl.when(j + NBUF < CPT)
            def _next():
                for b in range(NBUF):
                    pltpu.make_async_copy(row_vms[b],
                                          agg.at[dst_vms[b].at[0]],
                                          ssems[b]).wait()
                    pltpu.async_copy(u_hbm.at[src_vm.at[j + NBUF + b]],
                                     row_vms[b], gsems[b])

            @pl.when(j + NBUF >= CPT)
            def _last():
                for b in range(NBUF):
                    pltpu.make_async_copy(row_vms[b],
                                          agg.at[dst_vms[b].at[0]],
                                          ssems[b]).wait()

        plsc.subcore_barrier()

        @pl.loop(0, NBLK)
        def _upd(b):
            off = rbase + b * BLK
            pltpu.sync_copy(agg.at[pl.ds(off, BLK)], blk)
            pltpu.sync_copy(h_hbm.at[pl.ds(hbase + b * BLK, BLK)], hblk)
            pltpu.sync_copy(dinv_sp.at[pl.ds(off, BLK)], dvblk)

            @pl.loop(0, BLK)
            def _row(r):
                dvec = dvblk[r, :]
                svec = (1.0 - ALPHA) * dvec
                for c in range(DH // 16):
                    sl = pl.ds(c * 16, 16)
                    z = blk[r, sl] * svec + ALPHA * hblk[r, sl]
                    blk[r, sl] = z * dvec

            pltpu.sync_copy(blk, u_hbm.at[pl.ds(hbase + b * BLK, BLK)])
            pltpu.sync_copy(zb, agg.at[pl.ds(off, BLK)])

        plsc.subcore_barrier()

    @pl.loop(0, NBLK)
    def _fin(b):
        off = rbase + b * BLK
        pltpu.sync_copy(u_hbm.at[pl.ds(hbase + b * BLK, BLK)], hblk)
        pltpu.sync_copy(dinv_sp.at[pl.ds(off, BLK)], dvblk)

        @pl.loop(0, BLK)
        def _row(r):
            rvec = 1.0 / dvblk[r, :]
            for c in range(DH // 16):
                sl = pl.ds(c * 16, 16)
                hblk[r, sl] = hblk[r, sl] * rvec

        pltpu.sync_copy(hblk, z_hbm.at[pl.ds(hbase + b * BLK, BLK)])


_sc_call = pl.kernel(
    _sc_body,
    out_type=(jax.ShapeDtypeStruct((2 * N_PAD, DH), jnp.float32),
              jax.ShapeDtypeStruct((2 * N_PAD, DH), jnp.float32)),
    mesh=plsc.VectorSubcoreMesh(core_axis_name="c", subcore_axis_name="s"),
    compiler_params=pltpu.CompilerParams(use_tc_tiling_on_sc=False),
    scratch_types=[
        pltpu.VMEM_SHARED((N_PAD, DH), jnp.float32),
        pltpu.VMEM_SHARED((N_PAD, 16), jnp.float32),
        pltpu.VMEM((CPT, CHUNK), jnp.int32),
        tuple(pltpu.VMEM((1, CHUNK), jnp.int32)
              for _ in range(NBUF)),
        tuple(pltpu.VMEM((CHUNK, DH), jnp.float32)
              for _ in range(NBUF)),
        pltpu.VMEM((CHUNK, 16), jnp.float32),
        pltpu.VMEM((BLK, DH), jnp.float32),
        pltpu.VMEM((BLK, 16), jnp.float32),
        pltpu.VMEM((BLK, DH), jnp.float32),
        pltpu.VMEM((BLK, DH), jnp.float32),
        pltpu.VMEM((BLK, 16), jnp.float32),
        tuple(pltpu.SemaphoreType.DMA
              for _ in range(NBUF)),
        tuple(pltpu.SemaphoreType.DMA
              for _ in range(NBUF)),
    ],
)


def kernel(x, edge_index, W1, b1, W2, b2):
    x_pad = jnp.pad(x, ((0, N_PAD - N), (0, 0)))
    h_pad = _mlp(x_pad, W1, b1, W2, b2)
    h_split = jnp.concatenate([h_pad[:, :DH], h_pad[:, DH:]], axis=0)

    src = edge_index[0]
    dst = edge_index[1]
    loop_idx = jnp.arange(N, dtype=jnp.int32)
    pad_idx = N + (jnp.arange(E_PAD - E_FULL, dtype=jnp.int32) % 16)
    srcs = jnp.concatenate([src, loop_idx, pad_idx]).reshape(NT * CPT, CHUNK)
    dsts = jnp.concatenate([dst, loop_idx, pad_idx]).reshape(NT * CPT, CHUNK)
    gsrcs = jnp.concatenate([srcs, srcs + N_PAD], axis=0)

    ones16 = jnp.ones((CHUNK, 16), jnp.float32)
    zrow = jnp.zeros((BLK, DH), jnp.float32)
    z16 = jnp.zeros((BLK, 16), jnp.float32)

    z_split, _ = _sc_call(h_split, gsrcs, dsts, ones16, zrow, z16)
    return jnp.concatenate([z_split[:N], z_split[N_PAD:N_PAD + N]], axis=1)


# speedup vs baseline: 15.4731x; 1.1481x over previous
import jax
import jax.numpy as jnp
from jax import lax
from jax.experimental import pallas as pl
from jax.experimental.pallas import tpu as pltpu
from jax.experimental.pallas import tpu_sc as plsc

N = 10000
D = 128
DH = D // 2
E = 320000
K = 10
ALPHA = 0.1

NT = 16
N_PAD = 10240
RPT = N_PAD // NT
BLK = 32
NBLK = RPT // BLK
CHUNK = 128
E_FULL = E + N
CPT = (-(-E_FULL // (NT * CHUNK)) + 7) // 8 * 8
E_PAD = NT * CPT * CHUNK


def _mlp_body(x_ref, w1_ref, b1_ref, w2_ref, b2_ref, o_ref):
    h = jnp.dot(x_ref[...], w1_ref[...], preferred_element_type=jnp.float32)
    h = jnp.maximum(h + b1_ref[...], 0.0)
    o = jnp.dot(h, w2_ref[...], preferred_element_type=jnp.float32)
    o_ref[...] = o + b2_ref[...]


def _mlp(x_pad, W1, b1, W2, b2):
    BM = 1024
    return pl.pallas_call(
        _mlp_body,
        grid=(N_PAD // BM,),
        in_specs=[
            pl.BlockSpec((BM, D), lambda i: (i, 0)),
            pl.BlockSpec((D, D), lambda i: (0, 0)),
            pl.BlockSpec((1, D), lambda i: (0, 0)),
            pl.BlockSpec((D, D), lambda i: (0, 0)),
            pl.BlockSpec((1, D), lambda i: (0, 0)),
        ],
        out_specs=pl.BlockSpec((BM, D), lambda i: (i, 0)),
        out_shape=jax.ShapeDtypeStruct((N_PAD, D), jnp.float32),
    )(x_pad, W1, b1.reshape(1, D), W2, b2.reshape(1, D))


NBUF = 3


def _sc_body(h_hbm, gsrcs_hbm, dsts_hbm, ones_hbm, zrow_hbm, z16_hbm,
             z_hbm, u_hbm,
             agg, dinv_sp, src_vm, dst_vm, row_vms,
             ones_vm, zb, zb16, blk, hblk, dvblk, gsems, ssems):
    cid = lax.axis_index("c")
    tid = lax.axis_index("s")
    ebase = tid * CPT
    rbase = tid * RPT
    hbase = cid * N_PAD + rbase

    pltpu.sync_copy(gsrcs_hbm.at[pl.ds(cid * NT * CPT + ebase, CPT)], src_vm)
    pltpu.sync_copy(dsts_hbm.at[pl.ds(ebase, CPT)], dst_vm)
    pltpu.sync_copy(ones_hbm, ones_vm)
    pltpu.sync_copy(zrow_hbm, zb)
    pltpu.sync_copy(z16_hbm, zb16)

    @pl.loop(0, NBLK)
    def _zero(b):
        off = rbase + b * BLK
        pltpu.sync_copy(zb16, dinv_sp.at[pl.ds(off, BLK)])
        pltpu.sync_copy(zb, agg.at[pl.ds(off, BLK)])

    plsc.subcore_barrier()

    @pl.loop(0, CPT, step=NBUF)
    def _deg(j):
        cps = [pltpu.async_copy(ones_vm, dinv_sp.at[dst_vm.at[j + b]],
                                ssems[b], add=True) for b in range(NBUF)]
        for cp in cps:
            cp.wait()

    plsc.subcore_barrier()

    @pl.loop(0, NBLK)
    def _dinv(b):
        off = rbase + b * BLK
        pltpu.sync_copy(dinv_sp.at[pl.ds(off, BLK)], dvblk)

        @pl.loop(0, BLK // 16)
        def _grp(g):
            for i in range(16):
                r = g * 16 + i
                d = dvblk[r, :]
                di = lax.bitcast_convert_type(d, jnp.int32)
                y = lax.bitcast_convert_type(
                    jnp.int32(0x5F3759DF) - (di >> 1), jnp.float32)
                for _ in range(3):
                    y = y * (1.5 - 0.5 * d * y * y)
                dvblk[r, :] = y

        pltpu.sync_copy(dvblk, dinv_sp.at[pl.ds(off, BLK)])

    @pl.loop(0, NBLK)
    def _u0(b):
        off = rbase + b * BLK
        pltpu.sync_copy(h_hbm.at[pl.ds(hbase + b * BLK, BLK)], hblk)
        pltpu.sync_copy(dinv_sp.at[pl.ds(off, BLK)], dvblk)

        @pl.loop(0, BLK)
        def _row(r):
            dvec = dvblk[r, :]
            for c in range(DH // 16):
                sl = pl.ds(c * 16, 16)
                hblk[r, sl] = hblk[r, sl] * dvec

        pltpu.sync_copy(hblk, u_hbm.at[pl.ds(hbase + b * BLK, BLK)])

    plsc.subcore_barrier()

    @pl.loop(0, K)
    def _iter(_k):
        for b in range(NBUF):
            pltpu.async_copy(u_hbm.at[src_vm.at[b]], row_vms[b], gsems[b])

        @pl.loop(0, CPT, step=NBUF)
        def _edges(j):
            for b in range(NBUF):
                pltpu.make_async_copy(u_hbm.at[src_vm.at[j + b]],
                                      row_vms[b], gsems[b]).wait()
                pltpu.async_copy(row_vms[b], agg.at[dst_vm.at[j + b]],
                                 ssems[b], add=True)

            @pl.when(j + NBUF < CPT)
            def _next():
                for b in range(NBUF):
                    pltpu.make_async_copy(row_vms[b],
                                          agg.at[dst_vm.at[j + b]],
                                          ssems[b]).wait()
                    pltpu.async_copy(u_hbm.at[src_vm.at[j + NBUF + b]],
                                     row_vms[b], gsems[b])

            @pl.when(j + NBUF >= CPT)
            def _last():
                for b in range(NBUF):
                    pltpu.make_async_copy(row_vms[b],
                                          agg.at[dst_vm.at[j + b]],
                                          ssems[b]).wait()

        plsc.subcore_barrier()

        @pl.loop(0, NBLK)
        def _upd(b):
            off = rbase + b * BLK
            pltpu.sync_copy(agg.at[pl.ds(off, BLK)], blk)
            pltpu.sync_copy(h_hbm.at[pl.ds(hbase + b * BLK, BLK)], hblk)
            pltpu.sync_copy(dinv_sp.at[pl.ds(off, BLK)], dvblk)

            @pl.loop(0, BLK)
            def _row(r):
                dvec = dvblk[r, :]
                svec = (1.0 - ALPHA) * dvec
                for c in range(DH // 16):
                    sl = pl.ds(c * 16, 16)
                    z = blk[r, sl] * svec + ALPHA * hblk[r, sl]
                    blk[r, sl] = z * dvec

            pltpu.sync_copy(blk, u_hbm.at[pl.ds(hbase + b * BLK, BLK)])
            pltpu.sync_copy(zb, agg.at[pl.ds(off, BLK)])

        plsc.subcore_barrier()

    @pl.loop(0, NBLK)
    def _fin(b):
        off = rbase + b * BLK
        pltpu.sync_copy(u_hbm.at[pl.ds(hbase + b * BLK, BLK)], hblk)
        pltpu.sync_copy(dinv_sp.at[pl.ds(off, BLK)], dvblk)

        @pl.loop(0, BLK)
        def _row(r):
            rvec = 1.0 / dvblk[r, :]
            for c in range(DH // 16):
                sl = pl.ds(c * 16, 16)
                hblk[r, sl] = hblk[r, sl] * rvec

        pltpu.sync_copy(hblk, z_hbm.at[pl.ds(hbase + b * BLK, BLK)])


_sc_call = pl.kernel(
    _sc_body,
    out_type=(jax.ShapeDtypeStruct((2 * N_PAD, DH), jnp.float32),
              jax.ShapeDtypeStruct((2 * N_PAD, DH), jnp.float32)),
    mesh=plsc.VectorSubcoreMesh(core_axis_name="c", subcore_axis_name="s"),
    compiler_params=pltpu.CompilerParams(use_tc_tiling_on_sc=False),
    scratch_types=[
        pltpu.VMEM_SHARED((N_PAD, DH), jnp.float32),
        pltpu.VMEM_SHARED((N_PAD, 16), jnp.float32),
        pltpu.VMEM((CPT, CHUNK), jnp.int32),
        pltpu.VMEM((CPT, CHUNK), jnp.int32),
        tuple(pltpu.VMEM((CHUNK, DH), jnp.float32)
              for _ in range(NBUF)),
        pltpu.VMEM((CHUNK, 16), jnp.float32),
        pltpu.VMEM((BLK, DH), jnp.float32),
        pltpu.VMEM((BLK, 16), jnp.float32),
        pltpu.VMEM((BLK, DH), jnp.float32),
        pltpu.VMEM((BLK, DH), jnp.float32),
        pltpu.VMEM((BLK, 16), jnp.float32),
        tuple(pltpu.SemaphoreType.DMA
              for _ in range(NBUF)),
        tuple(pltpu.SemaphoreType.DMA
              for _ in range(NBUF)),
    ],
)


def kernel(x, edge_index, W1, b1, W2, b2):
    x_pad = jnp.pad(x, ((0, N_PAD - N), (0, 0)))
    h_pad = _mlp(x_pad, W1, b1, W2, b2)
    h_split = jnp.concatenate([h_pad[:, :DH], h_pad[:, DH:]], axis=0)

    src = edge_index[0]
    dst = edge_index[1]
    loop_idx = jnp.arange(N, dtype=jnp.int32)
    pad_idx = N + (jnp.arange(E_PAD - E_FULL, dtype=jnp.int32) % 16)
    srcs = jnp.concatenate([src, loop_idx, pad_idx]).reshape(NT * CPT, CHUNK)
    dsts = jnp.concatenate([dst, loop_idx, pad_idx]).reshape(NT * CPT, CHUNK)
    gsrcs = jnp.concatenate([srcs, srcs + N_PAD], axis=0)

    ones16 = jnp.ones((CHUNK, 16), jnp.float32)
    zrow = jnp.zeros((BLK, DH), jnp.float32)
    z16 = jnp.zeros((BLK, 16), jnp.float32)

    z_split, _ = _sc_call(h_split, gsrcs, dsts, ones16, zrow, z16)
    return jnp.concatenate([z_split[:N], z_split[N_PAD:N_PAD + N]], axis=1)


# speedup vs baseline: 15.8677x; 1.0255x over previous
import jax
import jax.numpy as jnp
from jax import lax
from jax.experimental import pallas as pl
from jax.experimental.pallas import tpu as pltpu
from jax.experimental.pallas import tpu_sc as plsc

N = 10000
D = 128
DH = D // 2
E = 320000
K = 10
ALPHA = 0.1

NT = 16
N_PAD = 10240
RPT = N_PAD // NT
BLK = 32
NBLK = RPT // BLK
CHUNK = 128
E_FULL = E + N
GRP = 12
SG = 2 * GRP
CPT = (-(-E_FULL // (NT * CHUNK)) + SG - 1) // SG * SG
NSG = CPT // SG
E_PAD = NT * CPT * CHUNK


def _mlp_body(x_ref, w1_ref, b1_ref, w2_ref, b2_ref, o_ref):
    h = jnp.dot(x_ref[...], w1_ref[...], preferred_element_type=jnp.float32)
    h = jnp.maximum(h + b1_ref[...], 0.0)
    o = jnp.dot(h, w2_ref[...], preferred_element_type=jnp.float32)
    o_ref[...] = o + b2_ref[...]


def _mlp(x_pad, W1, b1, W2, b2):
    BM = 1024
    return pl.pallas_call(
        _mlp_body,
        grid=(N_PAD // BM,),
        in_specs=[
            pl.BlockSpec((BM, D), lambda i: (i, 0)),
            pl.BlockSpec((D, D), lambda i: (0, 0)),
            pl.BlockSpec((1, D), lambda i: (0, 0)),
            pl.BlockSpec((D, D), lambda i: (0, 0)),
            pl.BlockSpec((1, D), lambda i: (0, 0)),
        ],
        out_specs=pl.BlockSpec((BM, D), lambda i: (i, 0)),
        out_shape=jax.ShapeDtypeStruct((N_PAD, D), jnp.float32),
    )(x_pad, W1, b1.reshape(1, D), W2, b2.reshape(1, D))


NBUF = 6


def _sc_body(h_hbm, gsrcs_hbm, dsts_hbm, ones_hbm, zrow_hbm, z16_hbm,
             z_hbm, u_hbm,
             agg, dinv_sp, sidx, didx, row_vms,
             ones_vm, zb, zb16, blk, hblk, dvblk, gsems, ssems, isems):
    cid = lax.axis_index("c")
    tid = lax.axis_index("s")
    ebase = tid * CPT
    rbase = tid * RPT
    hbase = cid * N_PAD + rbase
    gebase = cid * NT * CPT + ebase

    def _idx_fetch(gb, half):
        pltpu.async_copy(gsrcs_hbm.at[pl.ds(gebase + gb, GRP)], sidx[half],
                         isems[half])
        pltpu.async_copy(dsts_hbm.at[pl.ds(ebase + gb, GRP)], didx[half],
                         isems[2 + half])

    def _idx_wait(gb, half):
        pltpu.make_async_copy(gsrcs_hbm.at[pl.ds(gebase + gb, GRP)],
                              sidx[half], isems[half]).wait()
        pltpu.make_async_copy(dsts_hbm.at[pl.ds(ebase + gb, GRP)],
                              didx[half], isems[2 + half]).wait()

    pltpu.sync_copy(ones_hbm, ones_vm)
    pltpu.sync_copy(zrow_hbm, zb)
    pltpu.sync_copy(z16_hbm, zb16)

    @pl.loop(0, NBLK)
    def _zero(b):
        off = rbase + b * BLK
        pltpu.sync_copy(zb16, dinv_sp.at[pl.ds(off, BLK)])
        pltpu.sync_copy(zb, agg.at[pl.ds(off, BLK)])

    plsc.subcore_barrier()

    _idx_fetch(0, 0)

    @pl.loop(0, NSG)
    def _deg(s):
        base = s * SG
        for half in range(2):
            gb = base + half * GRP
            _idx_wait(gb, half)

            @pl.when(gb + GRP < CPT)
            def _pf():
                _idx_fetch(gb + GRP, 1 - half)

            dV = didx[half]
            for k in range(0, GRP, NBUF):
                cps = [pltpu.async_copy(ones_vm,
                                        dinv_sp.at[dV.at[k + b]],
                                        ssems[b], add=True)
                       for b in range(NBUF)]
                for cp in cps:
                    cp.wait()

    plsc.subcore_barrier()

    @pl.loop(0, NBLK)
    def _dinv(b):
        off = rbase + b * BLK
        pltpu.sync_copy(dinv_sp.at[pl.ds(off, BLK)], dvblk)

        @pl.loop(0, BLK // 16)
        def _grp(g):
            for i in range(16):
                r = g * 16 + i
                d = dvblk[r, :]
                di = lax.bitcast_convert_type(d, jnp.int32)
                y = lax.bitcast_convert_type(
                    jnp.int32(0x5F3759DF) - (di >> 1), jnp.float32)
                for _ in range(3):
                    y = y * (1.5 - 0.5 * d * y * y)
                dvblk[r, :] = y

        pltpu.sync_copy(dvblk, dinv_sp.at[pl.ds(off, BLK)])

    @pl.loop(0, NBLK)
    def _u0(b):
        off = rbase + b * BLK
        pltpu.sync_copy(h_hbm.at[pl.ds(hbase + b * BLK, BLK)], hblk)
        pltpu.sync_copy(dinv_sp.at[pl.ds(off, BLK)], dvblk)

        @pl.loop(0, BLK)
        def _row(r):
            dvec = dvblk[r, :]
            for c in range(DH // 16):
                sl = pl.ds(c * 16, 16)
                hblk[r, sl] = hblk[r, sl] * dvec

        pltpu.sync_copy(hblk, u_hbm.at[pl.ds(hbase + b * BLK, BLK)])

    plsc.subcore_barrier()

    @pl.loop(0, K)
    def _iter(_k):
        _idx_fetch(0, 0)

        @pl.loop(0, NSG)
        def _sg(s):
            base = s * SG
            for half in range(2):
                gb = base + half * GRP
                _idx_wait(gb, half)

                @pl.when(gb + GRP < CPT)
                def _pf():
                    _idx_fetch(gb + GRP, 1 - half)

                sV = sidx[half]
                dV = didx[half]
                for b in range(NBUF):
                    pltpu.async_copy(u_hbm.at[sV.at[b]], row_vms[b],
                                     gsems[b])
                for k in range(0, GRP, NBUF):
                    for b in range(NBUF):
                        pltpu.make_async_copy(u_hbm.at[sV.at[k + b]],
                                              row_vms[b], gsems[b]).wait()
                        pltpu.async_copy(row_vms[b], agg.at[dV.at[k + b]],
                                         ssems[b], add=True)
                    if k + NBUF < GRP:
                        for b in range(NBUF):
                            pltpu.make_async_copy(
                                row_vms[b], agg.at[dV.at[k + b]],
                                ssems[b]).wait()
                            pltpu.async_copy(u_hbm.at[sV.at[k + NBUF + b]],
                                             row_vms[b], gsems[b])
                    else:
                        for b in range(NBUF):
                            pltpu.make_async_copy(
                                row_vms[b], agg.at[dV.at[k + b]],
                                ssems[b]).wait()

        plsc.subcore_barrier()

        @pl.loop(0, NBLK)
        def _upd(b):
            off = rbase + b * BLK
            pltpu.sync_copy(agg.at[pl.ds(off, BLK)], blk)
            pltpu.sync_copy(h_hbm.at[pl.ds(hbase + b * BLK, BLK)], hblk)
            pltpu.sync_copy(dinv_sp.at[pl.ds(off, BLK)], dvblk)

            @pl.loop(0, BLK)
            def _row(r):
                dvec = dvblk[r, :]
                svec = (1.0 - ALPHA) * dvec
                for c in range(DH // 16):
                    sl = pl.ds(c * 16, 16)
                    z = blk[r, sl] * svec + ALPHA * hblk[r, sl]
                    blk[r, sl] = z * dvec

            pltpu.sync_copy(blk, u_hbm.at[pl.ds(hbase + b * BLK, BLK)])
            pltpu.sync_copy(zb, agg.at[pl.ds(off, BLK)])

        plsc.subcore_barrier()

    @pl.loop(0, NBLK)
    def _fin(b):
        off = rbase + b * BLK
        pltpu.sync_copy(u_hbm.at[pl.ds(hbase + b * BLK, BLK)], hblk)
        pltpu.sync_copy(dinv_sp.at[pl.ds(off, BLK)], dvblk)

        @pl.loop(0, BLK)
        def _row(r):
            rvec = 1.0 / dvblk[r, :]
            for c in range(DH // 16):
                sl = pl.ds(c * 16, 16)
                hblk[r, sl] = hblk[r, sl] * rvec

        pltpu.sync_copy(hblk, z_hbm.at[pl.ds(hbase + b * BLK, BLK)])


_sc_call = pl.kernel(
    _sc_body,
    out_type=(jax.ShapeDtypeStruct((2 * N_PAD, DH), jnp.float32),
              jax.ShapeDtypeStruct((2 * N_PAD, DH), jnp.float32)),
    mesh=plsc.VectorSubcoreMesh(core_axis_name="c", subcore_axis_name="s"),
    compiler_params=pltpu.CompilerParams(use_tc_tiling_on_sc=False),
    scratch_types=[
        pltpu.VMEM_SHARED((N_PAD, DH), jnp.float32),
        pltpu.VMEM_SHARED((N_PAD, 16), jnp.float32),
        tuple(pltpu.VMEM((GRP, CHUNK), jnp.int32)
              for _ in range(2)),
        tuple(pltpu.VMEM((GRP, CHUNK), jnp.int32)
              for _ in range(2)),
        tuple(pltpu.VMEM((CHUNK, DH), jnp.float32)
              for _ in range(NBUF)),
        pltpu.VMEM((CHUNK, 16), jnp.float32),
        pltpu.VMEM((BLK, DH), jnp.float32),
        pltpu.VMEM((BLK, 16), jnp.float32),
        pltpu.VMEM((BLK, DH), jnp.float32),
        pltpu.VMEM((BLK, DH), jnp.float32),
        pltpu.VMEM((BLK, 16), jnp.float32),
        tuple(pltpu.SemaphoreType.DMA
              for _ in range(NBUF)),
        tuple(pltpu.SemaphoreType.DMA
              for _ in range(NBUF)),
        tuple(pltpu.SemaphoreType.DMA
              for _ in range(4)),
    ],
)


def kernel(x, edge_index, W1, b1, W2, b2):
    x_pad = jnp.pad(x, ((0, N_PAD - N), (0, 0)))
    h_pad = _mlp(x_pad, W1, b1, W2, b2)
    h_split = jnp.concatenate([h_pad[:, :DH], h_pad[:, DH:]], axis=0)

    src = edge_index[0]
    dst = edge_index[1]
    loop_idx = jnp.arange(N, dtype=jnp.int32)
    pad_idx = N + (jnp.arange(E_PAD - E_FULL, dtype=jnp.int32) % 16)
    srcs = jnp.concatenate([src, loop_idx, pad_idx]).reshape(NT * CPT, CHUNK)
    dsts = jnp.concatenate([dst, loop_idx, pad_idx]).reshape(NT * CPT, CHUNK)
    gsrcs = jnp.concatenate([srcs, srcs + N_PAD], axis=0)

    ones16 = jnp.ones((CHUNK, 16), jnp.float32)
    zrow = jnp.zeros((BLK, DH), jnp.float32)
    z16 = jnp.zeros((BLK, 16), jnp.float32)

    z_split, _ = _sc_call(h_split, gsrcs, dsts, ones16, zrow, z16)
    return jnp.concatenate([z_split[:N], z_split[N_PAD:N_PAD + N]], axis=1)


# speedup vs baseline: 16.8192x; 1.0600x over previous
import jax
import jax.numpy as jnp
from jax import lax
from jax.experimental import pallas as pl
from jax.experimental.pallas import tpu as pltpu
from jax.experimental.pallas import tpu_sc as plsc

N = 10000
D = 128
DH = D // 2
E = 320000
K = 10
ALPHA = 0.1

NT = 16
N_PAD = 10240
RPT = N_PAD // NT
BLK = 64
NBLK = RPT // BLK
CHUNK = 128
E_FULL = E + N
GRP = 12
SG = 2 * GRP
CPT = (-(-E_FULL // (NT * CHUNK)) + SG - 1) // SG * SG
NSG = CPT // SG
E_PAD = NT * CPT * CHUNK


def _mlp_body(x_ref, w1_ref, b1_ref, w2_ref, b2_ref, o_ref):
    h = jnp.dot(x_ref[...], w1_ref[...], preferred_element_type=jnp.float32)
    h = jnp.maximum(h + b1_ref[...], 0.0)
    o = jnp.dot(h, w2_ref[...], preferred_element_type=jnp.float32)
    o_ref[...] = o + b2_ref[...]


def _mlp(x_pad, W1, b1, W2, b2):
    BM = 1024
    return pl.pallas_call(
        _mlp_body,
        grid=(N_PAD // BM,),
        in_specs=[
            pl.BlockSpec((BM, D), lambda i: (i, 0)),
            pl.BlockSpec((D, D), lambda i: (0, 0)),
            pl.BlockSpec((1, D), lambda i: (0, 0)),
            pl.BlockSpec((D, D), lambda i: (0, 0)),
            pl.BlockSpec((1, D), lambda i: (0, 0)),
        ],
        out_specs=pl.BlockSpec((BM, D), lambda i: (i, 0)),
        out_shape=jax.ShapeDtypeStruct((N_PAD, D), jnp.float32),
    )(x_pad, W1, b1.reshape(1, D), W2, b2.reshape(1, D))


NBUF = 6


def _sc_body(h_hbm, gsrcs_hbm, dsts_hbm, ones_hbm, zrow_hbm, z16_hbm,
             z_hbm, u_hbm,
             agg, dinv_sp, sidx, didx, row_vms,
             ones_vm, zb, zb16, blk, hblk, dvblk, gsems, ssems, isems):
    cid = lax.axis_index("c")
    tid = lax.axis_index("s")
    ebase = tid * CPT
    rbase = tid * RPT
    hbase = cid * N_PAD + rbase
    gebase = cid * NT * CPT + ebase

    def _idx_fetch(gb, half):
        pltpu.async_copy(gsrcs_hbm.at[pl.ds(gebase + gb, GRP)], sidx[half],
                         isems[half])
        pltpu.async_copy(dsts_hbm.at[pl.ds(ebase + gb, GRP)], didx[half],
                         isems[2 + half])

    def _idx_wait(gb, half):
        pltpu.make_async_copy(gsrcs_hbm.at[pl.ds(gebase + gb, GRP)],
                              sidx[half], isems[half]).wait()
        pltpu.make_async_copy(dsts_hbm.at[pl.ds(ebase + gb, GRP)],
                              didx[half], isems[2 + half]).wait()

    pltpu.sync_copy(ones_hbm, ones_vm)
    pltpu.sync_copy(zrow_hbm, zb)
    pltpu.sync_copy(z16_hbm, zb16)

    @pl.loop(0, NBLK)
    def _zero(b):
        off = rbase + b * BLK
        pltpu.sync_copy(zb16, dinv_sp.at[pl.ds(off, BLK)])
        pltpu.sync_copy(zb, agg.at[pl.ds(off, BLK)])

    plsc.subcore_barrier()

    _idx_fetch(0, 0)

    @pl.loop(0, NSG)
    def _deg(s):
        base = s * SG
        for half in range(2):
            gb = base + half * GRP
            _idx_wait(gb, half)

            @pl.when(gb + GRP < CPT)
            def _pf():
                _idx_fetch(gb + GRP, 1 - half)

            dV = didx[half]
            for k in range(0, GRP, NBUF):
                cps = [pltpu.async_copy(ones_vm,
                                        dinv_sp.at[dV.at[k + b]],
                                        ssems[b], add=True)
                       for b in range(NBUF)]
                for cp in cps:
                    cp.wait()

    plsc.subcore_barrier()

    @pl.loop(0, NBLK)
    def _dinv(b):
        off = rbase + b * BLK
        pltpu.sync_copy(dinv_sp.at[pl.ds(off, BLK)], dvblk)

        @pl.loop(0, BLK // 16)
        def _grp(g):
            for i in range(16):
                r = g * 16 + i
                d = dvblk[r, :]
                di = lax.bitcast_convert_type(d, jnp.int32)
                y = lax.bitcast_convert_type(
                    jnp.int32(0x5F3759DF) - (di >> 1), jnp.float32)
                for _ in range(3):
                    y = y * (1.5 - 0.5 * d * y * y)
                dvblk[r, :] = y

        pltpu.sync_copy(dvblk, dinv_sp.at[pl.ds(off, BLK)])

    @pl.loop(0, NBLK)
    def _u0(b):
        off = rbase + b * BLK
        pltpu.sync_copy(h_hbm.at[pl.ds(hbase + b * BLK, BLK)], hblk)
        pltpu.sync_copy(dinv_sp.at[pl.ds(off, BLK)], dvblk)

        @pl.loop(0, BLK)
        def _row(r):
            dvec = dvblk[r, :]
            for c in range(DH // 16):
                sl = pl.ds(c * 16, 16)
                hblk[r, sl] = hblk[r, sl] * dvec

        pltpu.sync_copy(hblk, u_hbm.at[pl.ds(hbase + b * BLK, BLK)])

    plsc.subcore_barrier()

    @pl.loop(0, K)
    def _iter(_k):
        _idx_fetch(0, 0)

        @pl.loop(0, NSG)
        def _sg(s):
            base = s * SG
            for half in range(2):
                gb = base + half * GRP
                _idx_wait(gb, half)

                @pl.when(gb + GRP < CPT)
                def _pf():
                    _idx_fetch(gb + GRP, 1 - half)

                sV = sidx[half]
                dV = didx[half]
                for b in range(NBUF):
                    pltpu.async_copy(u_hbm.at[sV.at[b]], row_vms[b],
                                     gsems[b])
                for k in range(0, GRP, NBUF):
                    for b in range(NBUF):
                        pltpu.make_async_copy(u_hbm.at[sV.at[k + b]],
                                              row_vms[b], gsems[b]).wait()
                        pltpu.async_copy(row_vms[b], agg.at[dV.at[k + b]],
                                         ssems[b], add=True)
                    if k + NBUF < GRP:
                        for b in range(NBUF):
                            pltpu.make_async_copy(
                                row_vms[b], agg.at[dV.at[k + b]],
                                ssems[b]).wait()
                            pltpu.async_copy(u_hbm.at[sV.at[k + NBUF + b]],
                                             row_vms[b], gsems[b])
                    else:
                        for b in range(NBUF):
                            pltpu.make_async_copy(
                                row_vms[b], agg.at[dV.at[k + b]],
                                ssems[b]).wait()

        plsc.subcore_barrier()

        @pl.loop(0, NBLK)
        def _upd(b):
            off = rbase + b * BLK
            pltpu.sync_copy(agg.at[pl.ds(off, BLK)], blk)
            pltpu.sync_copy(h_hbm.at[pl.ds(hbase + b * BLK, BLK)], hblk)
            pltpu.sync_copy(dinv_sp.at[pl.ds(off, BLK)], dvblk)

            @pl.loop(0, BLK)
            def _row(r):
                dvec = dvblk[r, :]
                svec = (1.0 - ALPHA) * dvec
                for c in range(DH // 16):
                    sl = pl.ds(c * 16, 16)
                    z = blk[r, sl] * svec + ALPHA * hblk[r, sl]
                    blk[r, sl] = z * dvec

            pltpu.sync_copy(blk, u_hbm.at[pl.ds(hbase + b * BLK, BLK)])
            pltpu.sync_copy(zb, agg.at[pl.ds(off, BLK)])

        plsc.subcore_barrier()

    @pl.loop(0, NBLK)
    def _fin(b):
        off = rbase + b * BLK
        pltpu.sync_copy(u_hbm.at[pl.ds(hbase + b * BLK, BLK)], hblk)
        pltpu.sync_copy(dinv_sp.at[pl.ds(off, BLK)], dvblk)

        @pl.loop(0, BLK)
        def _row(r):
            rvec = 1.0 / dvblk[r, :]
            for c in range(DH // 16):
                sl = pl.ds(c * 16, 16)
                hblk[r, sl] = hblk[r, sl] * rvec

        pltpu.sync_copy(hblk, z_hbm.at[pl.ds(hbase + b * BLK, BLK)])


_sc_call = pl.kernel(
    _sc_body,
    out_type=(jax.ShapeDtypeStruct((2 * N_PAD, DH), jnp.float32),
              jax.ShapeDtypeStruct((2 * N_PAD, DH), jnp.float32)),
    mesh=plsc.VectorSubcoreMesh(core_axis_name="c", subcore_axis_name="s"),
    compiler_params=pltpu.CompilerParams(use_tc_tiling_on_sc=False),
    scratch_types=[
        pltpu.VMEM_SHARED((N_PAD, DH), jnp.float32),
        pltpu.VMEM_SHARED((N_PAD, 16), jnp.float32),
        tuple(pltpu.VMEM((GRP, CHUNK), jnp.int32)
              for _ in range(2)),
        tuple(pltpu.VMEM((GRP, CHUNK), jnp.int32)
              for _ in range(2)),
        tuple(pltpu.VMEM((CHUNK, DH), jnp.float32)
              for _ in range(NBUF)),
        pltpu.VMEM((CHUNK, 16), jnp.float32),
        pltpu.VMEM((BLK, DH), jnp.float32),
        pltpu.VMEM((BLK, 16), jnp.float32),
        pltpu.VMEM((BLK, DH), jnp.float32),
        pltpu.VMEM((BLK, DH), jnp.float32),
        pltpu.VMEM((BLK, 16), jnp.float32),
        tuple(pltpu.SemaphoreType.DMA
              for _ in range(NBUF)),
        tuple(pltpu.SemaphoreType.DMA
              for _ in range(NBUF)),
        tuple(pltpu.SemaphoreType.DMA
              for _ in range(4)),
    ],
)


def kernel(x, edge_index, W1, b1, W2, b2):
    x_pad = jnp.pad(x, ((0, N_PAD - N), (0, 0)))
    h_pad = _mlp(x_pad, W1, b1, W2, b2)
    h_split = jnp.concatenate([h_pad[:, :DH], h_pad[:, DH:]], axis=0)

    src = edge_index[0]
    dst = edge_index[1]
    loop_idx = jnp.arange(N, dtype=jnp.int32)
    pad_idx = N + (jnp.arange(E_PAD - E_FULL, dtype=jnp.int32) % 16)
    srcs = jnp.concatenate([src, loop_idx, pad_idx]).reshape(NT * CPT, CHUNK)
    dsts = jnp.concatenate([dst, loop_idx, pad_idx]).reshape(NT * CPT, CHUNK)
    gsrcs = jnp.concatenate([srcs, srcs + N_PAD], axis=0)

    ones16 = jnp.ones((CHUNK, 16), jnp.float32)
    zrow = jnp.zeros((BLK, DH), jnp.float32)
    z16 = jnp.zeros((BLK, 16), jnp.float32)

    z_split, _ = _sc_call(h_split, gsrcs, dsts, ones16, zrow, z16)
    return jnp.concatenate([z_split[:N], z_split[N_PAD:N_PAD + N]], axis=1)
